# baseline (speedup 1.0000x reference)
import jax
import jax.numpy as jnp
from jax import lax
from jax.experimental import pallas as pl
from jax.experimental.pallas import tpu as pltpu

_EPS = 1e-20

_R = 128
_C = 100000
_B = 8192
_NB = (_C + _B - 1) // _B

_BIG_F32 = 1e9


def _body(x_ref, u_ref, out_ref, max_ref, idx_ref):
    p = pl.program_id(0)
    j = pl.program_id(1)

    col0 = (j * _B).astype(jnp.float32)
    iota_f = lax.broadcasted_iota(jnp.int32, (_R, _B), 1).astype(jnp.float32)
    gcol = col0 + iota_f

    @pl.when(p == 0)
    def _phase0():
        t = -jnp.log(u_ref[...] + _EPS) + _EPS
        f = jnp.exp(x_ref[...]) / t
        f = jnp.where(gcol < float(_C), f, -1.0)
        m = jnp.max(f, axis=1, keepdims=True)
        cand = jnp.min(
            jnp.where(f == m, gcol, _BIG_F32), axis=1, keepdims=True
        )

        @pl.when(j == 0)
        def _init():
            max_ref[...] = m
            idx_ref[...] = cand

        @pl.when(j > 0)
        def _acc():
            better = m > max_ref[...]
            max_ref[...] = jnp.where(better, m, max_ref[...])
            idx_ref[...] = jnp.where(better, cand, idx_ref[...])

    @pl.when(p == 1)
    def _phase1():
        out_ref[...] = (gcol == idx_ref[...]).astype(jnp.float32)


@jax.jit
def kernel(x, U):
    grid = (2, _NB)

    def in_map(p, j):
        return (0, jnp.where(p == 0, j, _NB - 1))

    def out_map(p, j):
        return (0, jnp.where(p == 0, 0, j))

    return pl.pallas_call(
        _body,
        grid=grid,
        in_specs=[
            pl.BlockSpec((_R, _B), in_map),
            pl.BlockSpec((_R, _B), in_map),
        ],
        out_specs=pl.BlockSpec((_R, _B), out_map),
        out_shape=jax.ShapeDtypeStruct((_R, _C), jnp.float32),
        scratch_shapes=[
            pltpu.VMEM((_R, 1), jnp.float32),
            pltpu.VMEM((_R, 1), jnp.float32),
        ],
        compiler_params=pltpu.CompilerParams(
            dimension_semantics=("arbitrary", "arbitrary"),
        ),
    )(x, U)
